# Initial kernel scaffold; baseline (speedup 1.0000x reference)
import jax
import jax.numpy as jnp
from jax.experimental import pallas as pl


def kernel(tokens, table):
    raise NotImplementedError("write your pallas kernel here")


# speedup vs baseline: 3.6681x; 3.6681x over previous
import functools
import math

import jax
import jax.numpy as jnp
from jax import lax
from jax.experimental import pallas as pl
from jax.experimental.pallas import tpu as pltpu
from jax.experimental.pallas import tpu_sc as plsc

_NC = 2
_NS = 16
_NW = _NC * _NS

_CHUNK = 1024
_STREAM = 128


def _prescale(table, scale):
    v, d = table.shape
    blk = 1000
    assert v % blk == 0

    def body(t_ref, o_ref):
        o_ref[...] = t_ref[...] * scale

    return pl.pallas_call(
        body,
        grid=(v // blk,),
        in_specs=[pl.BlockSpec((blk, d), lambda i: (i, 0))],
        out_specs=pl.BlockSpec((blk, d), lambda i: (i, 0)),
        out_shape=jax.ShapeDtypeStruct((v, d), jnp.float32),
    )(table)


@functools.partial(jax.jit, static_argnums=(2, 3))
def _gather(scaled, idx, n, d):
    n_per_w = n // _NW
    n_chunks = n_per_w // _CHUNK
    n_streams = _CHUNK // _STREAM
    mesh = plsc.VectorSubcoreMesh(core_axis_name="c", subcore_axis_name="s")

    @functools.partial(
        pl.kernel,
        out_type=jax.ShapeDtypeStruct((n, d), jnp.float32),
        mesh=mesh,
        compiler_params=pltpu.CompilerParams(use_tc_tiling_on_sc=False),
        scratch_types=[
            pltpu.VMEM((_CHUNK,), jnp.int32),
            pltpu.VMEM((_CHUNK, d), jnp.float32),
            pltpu.SemaphoreType.DMA,
        ],
    )
    def gather(table_hbm, idx_hbm, out_hbm, idx_v, rows_v, sem):
        wid = lax.axis_index("s") * _NC + lax.axis_index("c")
        base = wid * n_per_w

        def chunk_body(i, carry):
            off = base + i * _CHUNK
            pltpu.sync_copy(idx_hbm.at[pl.ds(off, _CHUNK)], idx_v)
            copies = [
                pltpu.async_copy(
                    table_hbm.at[idx_v.at[pl.ds(j * _STREAM, _STREAM)]],
                    rows_v.at[pl.ds(j * _STREAM, _STREAM)],
                    sem,
                )
                for j in range(n_streams)
            ]
            for cp in copies:
                cp.wait()
            pltpu.sync_copy(rows_v, out_hbm.at[pl.ds(off, _CHUNK)])
            return carry

        lax.fori_loop(0, n_chunks, chunk_body, 0)

    return gather(scaled, idx)


def kernel(tokens, table):
    b, s = tokens.shape
    v, d = table.shape
    n = b * s
    scale = math.sqrt(float(d))
    scaled = _prescale(table, scale)
    idx = tokens.reshape(n).astype(jnp.int32)
    out = _gather(scaled, idx, n, d)
    return out.reshape(b, s, d)


# speedup vs baseline: 5.1721x; 1.4100x over previous
import functools
import math

import jax
import jax.numpy as jnp
from jax import lax
from jax.experimental import pallas as pl
from jax.experimental.pallas import tpu as pltpu
from jax.experimental.pallas import tpu_sc as plsc

_NC = 2
_NS = 16
_NW = _NC * _NS

_CHUNK = 128
_GROUP = 1024
_CPG = _GROUP // _CHUNK


def _prescale_pad(table, scale):
    v, d = table.shape
    blk = 5000
    assert v % blk == 0

    def body(t_ref, o_ref):
        o_ref[:, :d] = t_ref[...] * scale
        o_ref[:, d:] = jnp.zeros((blk, 128 - d), jnp.float32)

    return pl.pallas_call(
        body,
        grid=(v // blk,),
        in_specs=[pl.BlockSpec((blk, d), lambda i: (i, 0))],
        out_specs=pl.BlockSpec((blk, 128), lambda i: (i, 0)),
        out_shape=jax.ShapeDtypeStruct((v, 128), jnp.float32),
    )(table)


def _gather(scaled, idx2d, n, d):
    n_per_w = n // _NW
    n_groups = n_per_w // _GROUP
    n_vregs = d // 16
    mesh = plsc.VectorSubcoreMesh(core_axis_name="c", subcore_axis_name="s")

    @functools.partial(
        pl.kernel,
        out_type=jax.ShapeDtypeStruct((n, d), jnp.float32),
        mesh=mesh,
        scratch_types=[
            pltpu.VMEM((8, 128), jnp.int32),
            pltpu.VMEM((_CHUNK, 128), jnp.float32),
            pltpu.VMEM((_CHUNK, 128), jnp.float32),
            pltpu.VMEM((_CHUNK, d), jnp.float32),
            pltpu.VMEM((_CHUNK, d), jnp.float32),
            pltpu.SemaphoreType.DMA,
            pltpu.SemaphoreType.DMA,
            pltpu.SemaphoreType.DMA,
            pltpu.SemaphoreType.DMA,
        ],
    )
    def gather(table_hbm, idx_hbm, out_hbm, idx_v, rv0, rv1, cv0, cv1,
               sg0, sg1, so0, so1):
        wid = lax.axis_index("s") * _NC + lax.axis_index("c")
        base = wid * n_per_w
        idx_base = wid * (n_per_w // 128)
        rv = (rv0, rv1)
        cv = (cv0, cv1)
        sg = (sg0, sg1)
        so = (so0, so1)

        def compact(src, dst):
            def crow(i, carry):
                r0 = i * 8
                for k in range(8):
                    for j in range(n_vregs):
                        dst[r0 + k, pl.ds(j * 16, 16)] = (
                            src[r0 + k, pl.ds(j * 16, 16)])
                return carry
            lax.fori_loop(0, _CHUNK // 8, crow, 0)

        def group_body(g, carry):
            pltpu.sync_copy(idx_hbm.at[pl.ds(idx_base + g * 8, 8)], idx_v)
            gather_cp = [None] * _CPG
            out_cp = [None] * _CPG
            gather_cp[0] = pltpu.async_copy(
                table_hbm.at[idx_v.at[0]], rv[0], sg[0])
            for q in range(_CPG):
                p = q % 2
                if q + 1 < _CPG:
                    gather_cp[q + 1] = pltpu.async_copy(
                        table_hbm.at[idx_v.at[q + 1]], rv[1 - p], sg[1 - p])
                gather_cp[q].wait()
                if q >= 2:
                    out_cp[q - 2].wait()
                compact(rv[p], cv[p])
                out_cp[q] = pltpu.async_copy(
                    cv[p],
                    out_hbm.at[pl.ds(base + g * _GROUP + q * _CHUNK, _CHUNK)],
                    so[p])
            out_cp[_CPG - 2].wait()
            out_cp[_CPG - 1].wait()
            return carry

        lax.fori_loop(0, n_groups, group_body, 0)

    return gather(scaled, idx2d)


def kernel(tokens, table):
    b, s = tokens.shape
    v, d = table.shape
    n = b * s
    scale = math.sqrt(float(d))
    scaled = _prescale_pad(table, scale)
    idx2d = tokens.reshape(n // 128, 128).astype(jnp.int32)
    out = _gather(scaled, idx2d, n, d)
    return out.reshape(b, s, d)
